# Initial kernel scaffold; baseline (speedup 1.0000x reference)
import jax
import jax.numpy as jnp
from jax.experimental import pallas as pl


def kernel(indices, spatial_embed):
    raise NotImplementedError("write your pallas kernel here")


# speedup vs baseline: 1.9326x; 1.9326x over previous
import jax
import jax.numpy as jnp
from jax import lax
from jax.experimental import pallas as pl
from jax.experimental.pallas import tpu as pltpu
from jax.experimental.pallas import tpu_sc as plsc

EMBED_DIM = 768
B = 32
N = 4096
NUM_IDX = B * N

NUM_WORKERS = 32
IDX_PER_TILE = NUM_IDX // NUM_WORKERS
CHUNK = 64
CHUNKS_PER_TILE = IDX_PER_TILE // CHUNK


def _body(table_hbm, idx_hbm, out_hbm,
          idx_v, buf0, buf1, gsem, wsem0, wsem1):
    cid = lax.axis_index("core")
    sid = lax.axis_index("subcore")
    wid = sid * 2 + cid
    base = wid * IDX_PER_TILE

    pltpu.sync_copy(idx_hbm.at[pl.ds(base, IDX_PER_TILE)], idx_v)

    bufs = (buf0, buf1)
    wsems = (wsem0, wsem1)

    @pl.loop(0, CHUNKS_PER_TILE, step=2)
    def _(c):
        for b in range(2):
            buf, wsem = bufs[b], wsems[b]
            cc = c + b
            @pl.when(cc >= 2)
            def _():
                pltpu.make_async_copy(
                    buf, out_hbm.at[pl.ds(base, CHUNK)], wsem
                ).wait()

            pltpu.async_copy(
                table_hbm.at[idx_v.at[pl.ds(cc * CHUNK, CHUNK)]], buf, gsem
            ).wait()
            pltpu.async_copy(
                buf, out_hbm.at[pl.ds(base + cc * CHUNK, CHUNK)], wsem
            )

    pltpu.make_async_copy(buf0, out_hbm.at[pl.ds(base, CHUNK)], wsem0).wait()
    pltpu.make_async_copy(buf1, out_hbm.at[pl.ds(base, CHUNK)], wsem1).wait()


def kernel(indices, spatial_embed):
    idx_flat = indices.reshape(NUM_IDX).astype(jnp.int32)
    mesh = plsc.VectorSubcoreMesh(
        core_axis_name="core", subcore_axis_name="subcore"
    )
    k = pl.kernel(
        _body,
        out_type=jax.ShapeDtypeStruct((NUM_IDX, EMBED_DIM), jnp.float32),
        mesh=mesh,
        scratch_types=[
            pltpu.VMEM((IDX_PER_TILE,), jnp.int32),
            pltpu.VMEM((CHUNK, EMBED_DIM), jnp.float32),
            pltpu.VMEM((CHUNK, EMBED_DIM), jnp.float32),
            pltpu.SemaphoreType.DMA,
            pltpu.SemaphoreType.DMA,
            pltpu.SemaphoreType.DMA,
        ],
    )
    out = k(spatial_embed, idx_flat)
    return out.reshape(B, N, EMBED_DIM)


# speedup vs baseline: 1.9327x; 1.0001x over previous
import jax
import jax.numpy as jnp
from jax import lax
from jax.experimental import pallas as pl
from jax.experimental.pallas import tpu as pltpu
from jax.experimental.pallas import tpu_sc as plsc

EMBED_DIM = 768
B = 32
N = 4096
NUM_IDX = B * N

NUM_WORKERS = 32
IDX_PER_TILE = NUM_IDX // NUM_WORKERS
CHUNK = 32
NBUF = 4
CHUNKS_PER_TILE = IDX_PER_TILE // CHUNK


def _body(table_hbm, idx_hbm, out_hbm, idx_v, *scratch):
    bufs = scratch[:NBUF]
    gsems = scratch[NBUF:2 * NBUF]
    wsems = scratch[2 * NBUF:3 * NBUF]

    cid = lax.axis_index("core")
    sid = lax.axis_index("subcore")
    wid = sid * 2 + cid
    base = wid * IDX_PER_TILE

    pltpu.sync_copy(idx_hbm.at[pl.ds(base, IDX_PER_TILE)], idx_v)

    @pl.loop(0, CHUNKS_PER_TILE, step=NBUF)
    def _(c):
        handles = []
        for b in range(NBUF):
            @pl.when(c >= NBUF)
            def _(b=b):
                pltpu.make_async_copy(
                    bufs[b], out_hbm.at[pl.ds(base, CHUNK)], wsems[b]
                ).wait()

            handles.append(pltpu.async_copy(
                table_hbm.at[idx_v.at[pl.ds((c + b) * CHUNK, CHUNK)]],
                bufs[b], gsems[b],
            ))
        for b in range(NBUF):
            handles[b].wait()
            pltpu.async_copy(
                bufs[b], out_hbm.at[pl.ds(base + (c + b) * CHUNK, CHUNK)],
                wsems[b],
            )

    for b in range(NBUF):
        pltpu.make_async_copy(
            bufs[b], out_hbm.at[pl.ds(base, CHUNK)], wsems[b]
        ).wait()


def kernel(indices, spatial_embed):
    idx_flat = indices.reshape(NUM_IDX).astype(jnp.int32)
    mesh = plsc.VectorSubcoreMesh(
        core_axis_name="core", subcore_axis_name="subcore"
    )
    k = pl.kernel(
        _body,
        out_type=jax.ShapeDtypeStruct((NUM_IDX, EMBED_DIM), jnp.float32),
        mesh=mesh,
        scratch_types=[
            pltpu.VMEM((IDX_PER_TILE,), jnp.int32),
            *[pltpu.VMEM((CHUNK, EMBED_DIM), jnp.float32)
              for _ in range(NBUF)],
            *[pltpu.SemaphoreType.DMA for _ in range(2 * NBUF)],
        ],
    )
    out = k(spatial_embed, idx_flat)
    return out.reshape(B, N, EMBED_DIM)
